# Initial kernel scaffold; baseline (speedup 1.0000x reference)
import jax
import jax.numpy as jnp
from jax.experimental import pallas as pl


def kernel(indices_, tables):
    raise NotImplementedError("write your pallas kernel here")


# speedup vs baseline: 66.6067x; 66.6067x over previous
import functools

import jax
import jax.numpy as jnp
from jax import lax
from jax.experimental import pallas as pl
from jax.experimental.pallas import tpu as pltpu
from jax.experimental.pallas import tpu_sc as plsc

NC = 2
NS = 16
NW = NC * NS

CH = 128
DMAS_PER_CHUNK = 4
C = CH * DMAS_PER_CHUNK


def _gather_kernel(B, d, b_per_w, n_c):
    mesh = plsc.VectorSubcoreMesh(
        core_axis_name="c", subcore_axis_name="s", num_cores=NC, num_subcores=NS
    )

    @functools.partial(
        pl.kernel,
        mesh=mesh,
        compiler_params=pltpu.CompilerParams(use_tc_tiling_on_sc=False),
        out_type=jax.ShapeDtypeStruct((B, d), jnp.float32),
        scratch_types=[
            pltpu.VMEM((b_per_w,), jnp.int32),
            pltpu.VMEM((2, C, d), jnp.float32),
            pltpu.SemaphoreType.DMA,
            pltpu.SemaphoreType.DMA,
        ],
    )
    def k(idx_hbm, table_hbm, out_hbm, idx_v, rows_v, gsem, wsem):
        wid = lax.axis_index("s") * NC + lax.axis_index("c")
        base = wid * b_per_w
        pltpu.sync_copy(idx_hbm.at[pl.ds(base, b_per_w)], idx_v)

        def gather_descs(j, slot):
            return [
                pltpu.make_async_copy(
                    table_hbm.at[idx_v.at[pl.ds(j * C + t * CH, CH)]],
                    rows_v.at[slot].at[pl.ds(t * CH, CH)],
                    gsem,
                )
                for t in range(DMAS_PER_CHUNK)
            ]

        def write_desc(j, slot):
            return pltpu.make_async_copy(
                rows_v.at[slot], out_hbm.at[pl.ds(base + j * C, C)], wsem
            )

        def body(j, carry):
            for dsc in gather_descs(j, 0):
                dsc.start()
            for dsc in gather_descs(j, 0):
                dsc.wait()
            write_desc(j, 0).start()
            write_desc(j, 0).wait()
            return carry

        lax.fori_loop(0, n_c, body, 0)

    return k


def kernel(indices_, tables):
    num_pages, page_size, d = tables.shape
    table = tables.reshape(num_pages * page_size, d)
    flat = indices_.reshape(-1).astype(jnp.int32)
    B = flat.shape[0]
    b_per_w = B // NW
    n_c = b_per_w // C

    out = _gather_kernel(B, d, b_per_w, n_c)(flat, table)
    return out.reshape(indices_.shape[0], indices_.shape[1], d)


# speedup vs baseline: 68.1606x; 1.0233x over previous
import functools

import jax
import jax.numpy as jnp
from jax import lax
from jax.experimental import pallas as pl
from jax.experimental.pallas import tpu as pltpu
from jax.experimental.pallas import tpu_sc as plsc

NC = 2
NS = 16
NW = NC * NS

CH = 128
DMAS_PER_CHUNK = 2
C = CH * DMAS_PER_CHUNK
NBUF = 4


def _gather_kernel(B, d, b_per_w, n_c):
    mesh = plsc.VectorSubcoreMesh(
        core_axis_name="c", subcore_axis_name="s", num_cores=NC, num_subcores=NS
    )

    @functools.partial(
        pl.kernel,
        mesh=mesh,
        compiler_params=pltpu.CompilerParams(use_tc_tiling_on_sc=False),
        out_type=jax.ShapeDtypeStruct((B, d), jnp.float32),
        scratch_types=[
            pltpu.VMEM((b_per_w,), jnp.int32),
            pltpu.VMEM((NBUF, C, d), jnp.float32),
            pltpu.SemaphoreType.DMA((NBUF,)),
            pltpu.SemaphoreType.DMA((NBUF,)),
        ],
    )
    def k(idx_hbm, table_hbm, out_hbm, idx_v, rows_v, gsem, wsem):
        wid = lax.axis_index("s") * NC + lax.axis_index("c")
        base = wid * b_per_w
        pltpu.sync_copy(idx_hbm.at[pl.ds(base, b_per_w)], idx_v)

        def gather_descs(j, slot):
            return [
                pltpu.make_async_copy(
                    table_hbm.at[idx_v.at[pl.ds(j * C + t * CH, CH)]],
                    rows_v.at[slot].at[pl.ds(t * CH, CH)],
                    gsem.at[slot],
                )
                for t in range(DMAS_PER_CHUNK)
            ]

        def write_desc(j, slot):
            return pltpu.make_async_copy(
                rows_v.at[slot], out_hbm.at[pl.ds(base + j * C, C)], wsem.at[slot]
            )

        for j0 in range(NBUF - 1):
            for dsc in gather_descs(j0, j0):
                dsc.start()

        def body(g, carry):
            for u in range(NBUF):
                j = g * NBUF + u
                pu = (u + NBUF - 1) % NBUF
                for dsc in gather_descs(j, u):
                    dsc.wait()
                write_desc(j, u).start()

                @pl.when(j > 0)
                def _retire_prev_write():
                    write_desc(j - 1, pu).wait()

                @pl.when(j + NBUF - 1 < n_c)
                def _prefetch():
                    for dsc in gather_descs(j + NBUF - 1, pu):
                        dsc.start()

            return carry

        lax.fori_loop(0, n_c // NBUF, body, 0)
        write_desc(n_c - 1, (n_c - 1) % NBUF).wait()

    return k


def kernel(indices_, tables):
    num_pages, page_size, d = tables.shape
    table = tables.reshape(num_pages * page_size, d)
    flat = indices_.reshape(-1).astype(jnp.int32)
    B = flat.shape[0]
    b_per_w = B // NW
    n_c = b_per_w // C
    assert B % NW == 0 and b_per_w % (C * NBUF) == 0

    out = _gather_kernel(B, d, b_per_w, n_c)(flat, table)
    return out.reshape(indices_.shape[0], indices_.shape[1], d)


# speedup vs baseline: 68.1895x; 1.0004x over previous
import functools

import jax
import jax.numpy as jnp
from jax import lax
from jax.experimental import pallas as pl
from jax.experimental.pallas import tpu as pltpu
from jax.experimental.pallas import tpu_sc as plsc

NC = 2
NS = 16
NW = NC * NS

NBUF = 4


def _gather_kernel(B0, S, d, b_per_w):
    n_c = b_per_w
    mesh = plsc.VectorSubcoreMesh(
        core_axis_name="c", subcore_axis_name="s", num_cores=NC, num_subcores=NS
    )

    @functools.partial(
        pl.kernel,
        mesh=mesh,
        compiler_params=pltpu.CompilerParams(use_tc_tiling_on_sc=False),
        out_type=jax.ShapeDtypeStruct((B0, S, d), jnp.float32),
        scratch_types=[
            pltpu.VMEM((b_per_w, S), jnp.int32),
            pltpu.VMEM((NBUF, S, d), jnp.float32),
            pltpu.SemaphoreType.DMA((NBUF,)),
            pltpu.SemaphoreType.DMA((NBUF,)),
        ],
    )
    def k(idx_hbm, table_hbm, out_hbm, idx_v, rows_v, gsem, wsem):
        table_flat = table_hbm.at[0]
        wid = lax.axis_index("s") * NC + lax.axis_index("c")
        base = wid * b_per_w
        pltpu.sync_copy(idx_hbm.at[pl.ds(base, b_per_w)], idx_v)

        def gather_desc(j, slot):
            return pltpu.make_async_copy(
                table_flat.at[idx_v.at[j]], rows_v.at[slot], gsem.at[slot]
            )

        def write_desc(j, slot):
            return pltpu.make_async_copy(
                rows_v.at[slot], out_hbm.at[base + j], wsem.at[slot]
            )

        for j0 in range(NBUF - 1):
            gather_desc(j0, j0).start()

        def body(g, carry):
            for u in range(NBUF):
                j = g * NBUF + u
                pu = (u + NBUF - 1) % NBUF
                gather_desc(j, u).wait()
                write_desc(j, u).start()

                @pl.when(j > 0)
                def _retire_prev_write():
                    write_desc(j - 1, pu).wait()

                @pl.when(j + NBUF - 1 < n_c)
                def _prefetch():
                    gather_desc(j + NBUF - 1, pu).start()

            return carry

        lax.fori_loop(0, n_c // NBUF, body, 0)
        write_desc(n_c - 1, (n_c - 1) % NBUF).wait()

    return k


def kernel(indices_, tables):
    num_pages, page_size, d = tables.shape
    B0, S = indices_.shape
    b_per_w = B0 // NW
    assert B0 % NW == 0 and b_per_w % NBUF == 0

    idx = indices_.astype(jnp.int32)
    return _gather_kernel(B0, S, d, b_per_w)(idx, tables)
